# Initial kernel scaffold; baseline (speedup 1.0000x reference)
import jax
import jax.numpy as jnp
from jax.experimental import pallas as pl


def kernel(x, weight):
    raise NotImplementedError("write your pallas kernel here")


# speedup vs baseline: 4.8825x; 4.8825x over previous
import functools

import jax
import jax.numpy as jnp
from jax import lax
from jax.experimental import pallas as pl
from jax.experimental.pallas import tpu as pltpu
from jax.experimental.pallas import tpu_sc as plsc

NC = 2
NS = 16
NW = NC * NS
L = 16

ROWS, COLS, D = 16384, 200, 5
B = ROWS * COLS
PER_W = B // NW
CHUNK = 4096
STEPS = PER_W // CHUNK
GROUPS = CHUNK // L


def _body(x_hbm, w_hbm, out_hbm, x_t, w_t, out_t):
    wid = lax.axis_index("s") * NC + lax.axis_index("c")
    pltpu.sync_copy(w_hbm, w_t)
    iota = lax.iota(jnp.int32, L)
    base_idx = wid * PER_W

    def step(s, _):
        off = base_idx + s * CHUNK
        pltpu.sync_copy(x_hbm.at[pl.ds(off, CHUNK)], x_t)

        def group(g, _):
            xv = x_t[pl.ds(g * L, L)]
            bases = xv * D
            pos = g * (D * L) + D * iota
            for c in range(D):
                vals = plsc.load_gather(w_t, [bases + c])
                plsc.store_scatter(out_t, [pos + c], vals)
            return 0

        lax.fori_loop(0, GROUPS, group, 0, unroll=False)
        pltpu.sync_copy(out_t, out_hbm.at[pl.ds(off * D, CHUNK * D)])
        return 0

    lax.fori_loop(0, STEPS, step, 0, unroll=False)


@jax.jit
def kernel(x, weight):
    xf = x.reshape(-1).astype(jnp.int32)
    wf = jnp.pad(weight.reshape(-1).astype(jnp.float32), (0, 80 - D * 14))
    mesh = plsc.VectorSubcoreMesh(core_axis_name="c", subcore_axis_name="s")
    run = pl.kernel(
        _body,
        out_type=jax.ShapeDtypeStruct((B * D,), jnp.float32),
        mesh=mesh,
        scratch_types=[
            pltpu.VMEM((CHUNK,), jnp.int32),
            pltpu.VMEM((80,), jnp.float32),
            pltpu.VMEM((CHUNK * D,), jnp.float32),
        ],
        compiler_params=pltpu.CompilerParams(needs_layout_passes=False),
    )
    out = run(xf, wf)
    return out.reshape(ROWS, COLS, D)


# speedup vs baseline: 5.2425x; 1.0737x over previous
import jax
import jax.numpy as jnp
from jax import lax
from jax.experimental import pallas as pl
from jax.experimental.pallas import tpu as pltpu
from jax.experimental.pallas import tpu_sc as plsc

NC = 2
NS = 16
NW = NC * NS
L = 16

ROWS, COLS, D = 16384, 200, 5
B = ROWS * COLS
PER_W = B // NW
CHUNK = 6400
STEPS = PER_W // CHUNK
GROUPS = CHUNK // L


def _body(x_hbm, w_hbm, out_hbm, x_t0, x_t1, w_t, out_t0, out_t1,
          si0, si1, so0, so1):
    wid = lax.axis_index("s") * NC + lax.axis_index("c")
    base_idx = wid * PER_W
    pltpu.sync_copy(w_hbm, w_t)
    iota5 = lax.iota(jnp.int32, L) * D
    x_bufs = (x_t0, x_t1)
    out_bufs = (out_t0, out_t1)
    sin = (si0, si1)
    sout = (so0, so1)

    def in_cp(step, b):
        return pltpu.make_async_copy(
            x_hbm.at[pl.ds(base_idx + step * CHUNK, CHUNK)], x_bufs[b], sin[b])

    def out_cp(step, b):
        return pltpu.make_async_copy(
            out_bufs[b],
            out_hbm.at[pl.ds((base_idx + step * CHUNK) * D, CHUNK * D)],
            sout[b])

    in_cp(0, 0).start()
    in_cp(1, 1).start()

    def stepfn(s2, _):
        for b in range(2):
            step = s2 * 2 + b
            in_cp(step, b).wait()

            @pl.when(step >= 2)
            def _wait_out():
                out_cp(step - 2, b).wait()

            x_t = x_bufs[b]
            out_t = out_bufs[b]

            @plsc.parallel_loop(0, GROUPS, unroll=8)
            def _group(g):
                xv = x_t[pl.ds(g * L, L)]
                bases = xv * D
                pos = g * (D * L) + iota5
                for c in range(D):
                    vals = plsc.load_gather(w_t, [bases + c])
                    plsc.store_scatter(out_t, [pos + c], vals)

            out_cp(step, b).start()

            @pl.when(step + 2 < STEPS)
            def _next_in():
                in_cp(step + 2, b).start()
        return 0

    lax.fori_loop(0, STEPS // 2, stepfn, 0, unroll=False)
    out_cp(STEPS - 2, 0).wait()
    out_cp(STEPS - 1, 1).wait()


@jax.jit
def kernel(x, weight):
    xf = x.reshape(-1).astype(jnp.int32)
    wf = jnp.pad(weight.reshape(-1).astype(jnp.float32), (0, 80 - D * 14))
    mesh = plsc.VectorSubcoreMesh(core_axis_name="c", subcore_axis_name="s")
    run = pl.kernel(
        _body,
        out_type=jax.ShapeDtypeStruct((B * D,), jnp.float32),
        mesh=mesh,
        scratch_types=[
            pltpu.VMEM((CHUNK,), jnp.int32),
            pltpu.VMEM((CHUNK,), jnp.int32),
            pltpu.VMEM((80,), jnp.float32),
            pltpu.VMEM((CHUNK * D,), jnp.float32),
            pltpu.VMEM((CHUNK * D,), jnp.float32),
            pltpu.SemaphoreType.DMA,
            pltpu.SemaphoreType.DMA,
            pltpu.SemaphoreType.DMA,
            pltpu.SemaphoreType.DMA,
        ],
        compiler_params=pltpu.CompilerParams(needs_layout_passes=False),
    )
    out = run(xf, wf)
    return out.reshape(ROWS, COLS, D)


# speedup vs baseline: 205.9227x; 39.2797x over previous
import jax
import jax.numpy as jnp
from jax import lax
from jax.experimental import pallas as pl
from jax.experimental.pallas import tpu as pltpu
from jax.experimental.pallas import tpu_sc as plsc

NC = 2
NS = 16
NW = NC * NS
L = 16

ROWS, COLS, D = 16384, 200, 5
PER_W = ROWS // NW
IB = 128
JB = 40
NJB = COLS // JB
NSTEP = (PER_W // IB) * NJB


def _body(xt_hbm, w_hbm, out_hbm, x_t0, x_t1, w_t, o_t0, o_t1,
          sx0, sx1, so0, so1):
    wid = lax.axis_index("s") * NC + lax.axis_index("c")
    i0w = wid * PER_W
    pltpu.sync_copy(w_hbm, w_t)
    x_bufs = (x_t0, x_t1)
    o_bufs = (o_t0, o_t1)
    sx = (sx0, sx1)
    so = (so0, so1)

    def x_cp(step, p):
        ib = step // NJB
        jb = step % NJB
        return pltpu.make_async_copy(
            xt_hbm.at[pl.ds(jb * JB, JB), pl.ds(i0w + ib * IB, IB)],
            x_bufs[p], sx[p])

    def o_cp(step, c, p):
        ib = step // NJB
        jb = step % NJB
        return pltpu.make_async_copy(
            o_bufs[p].at[c],
            out_hbm.at[pl.ds(c * COLS + jb * JB, JB),
                       pl.ds(i0w + ib * IB, IB)],
            so[p])

    x_cp(0, 0).start()
    x_cp(1, 1).start()

    def pair(s, _):
        for p in range(2):
            step = s * 2 + p
            x_t = x_bufs[p]
            o_t = o_bufs[p]
            x_cp(step, p).wait()

            @pl.when(step >= 2)
            def _drain():
                for c in range(D):
                    o_cp(step, c, p).wait()

            @plsc.parallel_loop(0, JB, unroll=2)
            def _grp(jrel):
                for g in range(IB // L):
                    xv = x_t[jrel, pl.ds(g * L, L)]
                    bases = xv * D
                    for c in range(D):
                        vals = plsc.load_gather(w_t, [bases + c])
                        o_t[c, jrel, pl.ds(g * L, L)] = vals

            for c in range(D):
                o_cp(step, c, p).start()

            @pl.when(step + 2 < NSTEP)
            def _next_x():
                x_cp(step + 2, p).start()
        return 0

    lax.fori_loop(0, NSTEP // 2, pair, 0, unroll=False)
    for c in range(D):
        o_cp(NSTEP - 2, c, 0).wait()
        o_cp(NSTEP - 1, c, 1).wait()


@jax.jit
def kernel(x, weight):
    xt = x.astype(jnp.int32).T
    wf = jnp.pad(weight.reshape(-1).astype(jnp.float32), (0, 80 - D * 14))
    mesh = plsc.VectorSubcoreMesh(core_axis_name="c", subcore_axis_name="s")
    run = pl.kernel(
        _body,
        out_type=jax.ShapeDtypeStruct((D * COLS, ROWS), jnp.float32),
        mesh=mesh,
        scratch_types=[
            pltpu.VMEM((JB, IB), jnp.int32),
            pltpu.VMEM((JB, IB), jnp.int32),
            pltpu.VMEM((80,), jnp.float32),
            pltpu.VMEM((D, JB, IB), jnp.float32),
            pltpu.VMEM((D, JB, IB), jnp.float32),
            pltpu.SemaphoreType.DMA,
            pltpu.SemaphoreType.DMA,
            pltpu.SemaphoreType.DMA,
            pltpu.SemaphoreType.DMA,
        ],
        compiler_params=pltpu.CompilerParams(
            needs_layout_passes=False, use_tc_tiling_on_sc=True),
    )
    out = run(xt, wf)
    return out.reshape(D, COLS, ROWS).transpose(2, 1, 0)


# speedup vs baseline: 206.0139x; 1.0004x over previous
import jax
import jax.numpy as jnp
from jax import lax
from jax.experimental import pallas as pl
from jax.experimental.pallas import tpu as pltpu
from jax.experimental.pallas import tpu_sc as plsc

NC = 2
NS = 16
NW = NC * NS
L = 16

ROWS, COLS, D = 16384, 200, 5
PER_W = ROWS // NW
IB = 128
JB = 40
NJB = COLS // JB
NSTEP = (PER_W // IB) * NJB


def _body(xt_hbm, w_hbm, out_hbm, x_t0, x_t1, w_t, o_t0, o_t1,
          sx0, sx1, so0, so1):
    wid = lax.axis_index("s") * NC + lax.axis_index("c")
    i0w = wid * PER_W
    pltpu.sync_copy(w_hbm, w_t)
    x_bufs = (x_t0, x_t1)
    o_bufs = (o_t0, o_t1)
    sx = (sx0, sx1)
    so = (so0, so1)

    def x_cp(step, p):
        ib = step // NJB
        jb = step % NJB
        return pltpu.make_async_copy(
            xt_hbm.at[pl.ds(jb * JB, JB), pl.ds(i0w + ib * IB, IB)],
            x_bufs[p], sx[p])

    def o_cp(step, c, p):
        ib = step // NJB
        jb = step % NJB
        return pltpu.make_async_copy(
            o_bufs[p].at[c],
            out_hbm.at[pl.ds(c * COLS + jb * JB, JB),
                       pl.ds(i0w + ib * IB, IB)],
            so[p])

    x_cp(0, 0).start()
    x_cp(1, 1).start()

    def pair(s, _):
        for p in range(2):
            step = s * 2 + p
            x_t = x_bufs[p]
            o_t = o_bufs[p]
            x_cp(step, p).wait()

            @pl.when(step >= 2)
            def _drain():
                for c in range(D):
                    o_cp(step, c, p).wait()

            @plsc.parallel_loop(0, JB, unroll=2)
            def _grp(jrel):
                for g in range(IB // L):
                    xv = x_t[jrel, pl.ds(g * L, L)]
                    bases = xv * D
                    for c in range(D):
                        vals = plsc.load_gather(w_t, [bases + c])
                        o_t[c, jrel, pl.ds(g * L, L)] = vals

            for c in range(D):
                o_cp(step, c, p).start()

            @pl.when(step + 2 < NSTEP)
            def _next_x():
                x_cp(step + 2, p).start()
        return 0

    lax.fori_loop(0, NSTEP // 2, pair, 0, unroll=False)
    for c in range(D):
        o_cp(NSTEP - 2, c, 0).wait()
        o_cp(NSTEP - 1, c, 1).wait()


@jax.jit
def kernel(x, weight):
    xt = x.astype(jnp.int32).T
    wf = weight.reshape(-1).astype(jnp.float32)
    mesh = plsc.VectorSubcoreMesh(core_axis_name="c", subcore_axis_name="s")
    run = pl.kernel(
        _body,
        out_type=jax.ShapeDtypeStruct((D * COLS, ROWS), jnp.float32),
        mesh=mesh,
        scratch_types=[
            pltpu.VMEM((JB, IB), jnp.int32),
            pltpu.VMEM((JB, IB), jnp.int32),
            pltpu.VMEM((70,), jnp.float32),
            pltpu.VMEM((D, JB, IB), jnp.float32),
            pltpu.VMEM((D, JB, IB), jnp.float32),
            pltpu.SemaphoreType.DMA,
            pltpu.SemaphoreType.DMA,
            pltpu.SemaphoreType.DMA,
            pltpu.SemaphoreType.DMA,
        ],
        compiler_params=pltpu.CompilerParams(
            needs_layout_passes=False, use_tc_tiling_on_sc=True),
    )
    out = run(xt, wf)
    return out.reshape(D, COLS, ROWS).transpose(2, 1, 0)


# speedup vs baseline: 221.0152x; 1.0728x over previous
import jax
import jax.numpy as jnp
from jax import lax
from jax.experimental import pallas as pl
from jax.experimental.pallas import tpu as pltpu
from jax.experimental.pallas import tpu_sc as plsc

NC = 2
NS = 16
NW = NC * NS
L = 16

ROWS, COLS, D = 16384, 200, 5
PER_W = ROWS // NW
IB = 128
JB = 40
NJB = COLS // JB
NSTEP = (PER_W // IB) * NJB


def _body(xt_hbm, w_hbm, out_hbm, x_t0, x_t1, w_t, o_t0, o_t1,
          sx0, sx1, so0, so1):
    wid = lax.axis_index("s") * NC + lax.axis_index("c")
    i0w = wid * PER_W
    pltpu.sync_copy(w_hbm, w_t)
    cvecs = [jnp.full((L,), c, jnp.int32) for c in range(D)]
    x_bufs = (x_t0, x_t1)
    o_bufs = (o_t0, o_t1)
    sx = (sx0, sx1)
    so = (so0, so1)

    def x_cp(step, p):
        ib = step // NJB
        jb = step % NJB
        return pltpu.make_async_copy(
            xt_hbm.at[pl.ds(jb * JB, JB), pl.ds(i0w + ib * IB, IB)],
            x_bufs[p], sx[p])

    def o_cp(step, c, p):
        ib = step // NJB
        jb = step % NJB
        return pltpu.make_async_copy(
            o_bufs[p].at[c],
            out_hbm.at[pl.ds(c * COLS + jb * JB, JB),
                       pl.ds(i0w + ib * IB, IB)],
            so[p])

    x_cp(0, 0).start()
    x_cp(1, 1).start()

    def pair(s, _):
        for p in range(2):
            step = s * 2 + p
            x_t = x_bufs[p]
            o_t = o_bufs[p]
            x_cp(step, p).wait()

            @pl.when(step >= 2)
            def _drain():
                for c in range(D):
                    o_cp(step, c, p).wait()

            @plsc.parallel_loop(0, JB, unroll=2)
            def _grp(jrel):
                for g in range(IB // L):
                    xv = x_t[jrel, pl.ds(g * L, L)]
                    for c in range(D):
                        vals = plsc.load_gather(w_t, [cvecs[c], xv])
                        o_t[c, jrel, pl.ds(g * L, L)] = vals

            for c in range(D):
                o_cp(step, c, p).start()

            @pl.when(step + 2 < NSTEP)
            def _next_x():
                x_cp(step + 2, p).start()
        return 0

    lax.fori_loop(0, NSTEP // 2, pair, 0, unroll=False)
    for c in range(D):
        o_cp(NSTEP - 2, c, 0).wait()
        o_cp(NSTEP - 1, c, 1).wait()


@jax.jit
def kernel(x, weight):
    xt = x.astype(jnp.int32).T
    wt = weight.astype(jnp.float32).T
    mesh = plsc.VectorSubcoreMesh(core_axis_name="c", subcore_axis_name="s")
    run = pl.kernel(
        _body,
        out_type=jax.ShapeDtypeStruct((D * COLS, ROWS), jnp.float32),
        mesh=mesh,
        scratch_types=[
            pltpu.VMEM((JB, IB), jnp.int32),
            pltpu.VMEM((JB, IB), jnp.int32),
            pltpu.VMEM((D, 14), jnp.float32),
            pltpu.VMEM((D, JB, IB), jnp.float32),
            pltpu.VMEM((D, JB, IB), jnp.float32),
            pltpu.SemaphoreType.DMA,
            pltpu.SemaphoreType.DMA,
            pltpu.SemaphoreType.DMA,
            pltpu.SemaphoreType.DMA,
        ],
        compiler_params=pltpu.CompilerParams(
            needs_layout_passes=False, use_tc_tiling_on_sc=True),
    )
    out = run(xt, wt)
    return out.reshape(D, COLS, ROWS).transpose(2, 1, 0)
